# baseline (speedup 1.0000x reference)
import functools

import numpy as np
import jax
import jax.numpy as jnp
from jax import lax
from jax.experimental import pallas as pl

_CONNECT = [(11, 12), (12, 13), (13, 14), (14, 15), (13, 25), (25, 26),
            (26, 27), (27, 29), (29, 30), (13, 17), (17, 18), (18, 19),
            (19, 21), (21, 22), (1, 2), (2, 3), (3, 4), (4, 5), (6, 7),
            (7, 8), (8, 9), (9, 10)]
_CHILD = [c for (_, c) in _CONNECT]
_ROOTS = (0, 1, 6, 11)
_IGNORE = (16, 20, 23, 24, 28, 31)
_EQUAL = (13, 19, 22, 13, 27, 30)

_parent = {c: p for (p, c) in _CONNECT}
_bone_of_child = {c: e for e, c in enumerate(_CHILD)}
_eq_map = dict(zip(_IGNORE, _EQUAL))


def _path_and_root(j):
    bones = []
    while j not in _ROOTS:
        bones.append(_bone_of_child[j])
        j = _parent[j]
    return bones, j


_A = np.zeros((22, 32), np.float32)
_RT = np.zeros((32,), np.int64)
for _j in range(32):
    _bones, _r = _path_and_root(_eq_map.get(_j, _j))
    _RT[_j] = _r
    for _e in _bones:
        _A[_e, _j] = 1.0

_W0 = np.zeros((22, 96), np.float32); _W0[:, 0::3] = _A
_W1 = np.zeros((22, 96), np.float32); _W1[:, 1::3] = _A
_W2 = np.zeros((22, 96), np.float32); _W2[:, 2::3] = _A

_C = np.zeros((96, 96), np.float32)
for _j in range(32):
    for _c in range(3):
        _C[3 * int(_RT[_j]) + _c, 3 * _j + _c] = 1.0

_SR = np.zeros((66, 22), np.float32)
_ST = np.zeros((66, 22), np.float32)
_SP = np.zeros((66, 22), np.float32)
for _e in range(22):
    _SR[3 * _e + 0, _e] = 1.0
    _ST[3 * _e + 1, _e] = 1.0
    _SP[3 * _e + 2, _e] = 1.0

_BATCH_PER_BLK = 8
_T = 100
_ROWS_PER_BLK = _BATCH_PER_BLK * _T

_P = np.zeros((_ROWS_PER_BLK, _BATCH_PER_BLK), np.float32)
for _i in range(_ROWS_PER_BLK):
    _P[_i, _i // _T] = 1.0

_INV_PI = 0.31830987334251404
_PI_HI = 3.140625
_PI_LO = float(np.float32(np.pi - 3.140625))
_SINC = (0.999999997000454, -0.16666659969977798, 0.008333097548004268,
         -0.0001981248476825909, 2.612900350327724e-06)
_COSC = (0.9999999998456127, -0.4999999951142109, 0.04166664187638779,
         -0.0013888432330831527, 2.4763766616282726e-05,
         -2.611494974122714e-07)


def _sincos(t):
    u = t * _INV_PI
    bias = jnp.where(u >= 0, 0.5, -0.5).astype(jnp.float32)
    n = (u + bias).astype(jnp.int32)
    nf = n.astype(jnp.float32)
    r = (t - nf * _PI_HI) - nf * _PI_LO
    r2 = r * r
    s = jnp.float32(_SINC[4])
    for k in (3, 2, 1, 0):
        s = s * r2 + jnp.float32(_SINC[k])
    s = s * r
    c = jnp.float32(_COSC[5])
    for k in (4, 3, 2, 1, 0):
        c = c * r2 + jnp.float32(_COSC[k])
    sgn = jnp.where((n & 1) == 0, 1.0, -1.0).astype(jnp.float32)
    return s * sgn, c * sgn


def _tc_body(pred_ref, obs_ref, sr_ref, st_ref, sp_ref, w0_ref, w1_ref,
             w2_ref, c_ref, p_ref, out_ref):
    p = pred_ref[...]
    f32 = jnp.float32
    r = jnp.dot(p, sr_ref[...], preferred_element_type=f32)
    th = jnp.dot(p, st_ref[...], preferred_element_type=f32)
    ph = jnp.dot(p, sp_ref[...], preferred_element_type=f32)
    sp_, cp_ = _sincos(ph)
    st_, ct_ = _sincos(th)
    rsp = r * sp_
    x = rsp * ct_
    y = rsp * st_
    z = r * cp_
    oc = jnp.dot(obs_ref[...], c_ref[...], preferred_element_type=f32)
    rep = jnp.dot(p_ref[...], oc, preferred_element_type=f32)
    out = (jnp.dot(x, w0_ref[...], preferred_element_type=f32)
           + jnp.dot(z, w1_ref[...], preferred_element_type=f32)
           + jnp.dot(y, w2_ref[...], preferred_element_type=f32)
           + rep)
    out_ref[...] = out


def _tc_kernel(observed_pose, pred_pose, interpret=False):
    B, T, D = pred_pose.shape
    pred_flat = pred_pose.reshape(B * T, D)
    obs_last = observed_pose[:, -1, :]
    n_blocks = (B * T) // _ROWS_PER_BLK
    full = lambda shp: pl.BlockSpec(shp, lambda i: (0, 0))
    out = pl.pallas_call(
        _tc_body,
        grid=(n_blocks,),
        in_specs=[
            pl.BlockSpec((_ROWS_PER_BLK, 66), lambda i: (i, 0)),
            pl.BlockSpec((_BATCH_PER_BLK, 96), lambda i: (i, 0)),
            full((66, 22)), full((66, 22)), full((66, 22)),
            full((22, 96)), full((22, 96)), full((22, 96)),
            full((96, 96)), full((_ROWS_PER_BLK, _BATCH_PER_BLK)),
        ],
        out_specs=pl.BlockSpec((_ROWS_PER_BLK, 96), lambda i: (i, 0)),
        out_shape=jax.ShapeDtypeStruct((B * T, 96), jnp.float32),
        interpret=interpret,
    )(pred_flat, obs_last, _SR, _ST, _SP, _W0, _W1, _W2, _C, _P)
    return out.reshape(B, T, 96)


def kernel(observed_pose, pred_pose):
    return _tc_kernel(observed_pose, pred_pose)
